# baseline (speedup 1.0000x reference)
import jax
import jax.numpy as jnp
from jax.experimental import pallas as pl
from jax.experimental.pallas import tpu as pltpu

N = 1024
NB = 4
B = N // NB

_TDIMS = (((0,), (0,)), ((), ()))


def _fused_kernel(adj_hbm, emb_ref, w1_ref, b1_ref, w2_ref, b2_ref,
                  fc1w_ref, fc1b_ref, fc2w_ref, fc2b_ref,
                  x_out_hbm, a2_out_ref, adj_v, x_stage, sem, xsem):
    for i in range(NB):
        pltpu.make_async_copy(adj_hbm.at[pl.ds(i * B, B), :],
                              adj_v.at[pl.ds(i * B, B), :],
                              sem.at[i]).start()

    embT = jnp.transpose(emb_ref[...])
    xtT = jax.lax.dot_general(w1_ref[...], embT, _TDIMS,
                              preferred_element_type=jnp.float32)

    ones_b = jnp.ones((1, B), jnp.float32)
    deg = jnp.full((1, N), 2.0, jnp.float32)
    for i in range(NB):
        pltpu.make_async_copy(adj_hbm.at[pl.ds(i * B, B), :],
                              adj_v.at[pl.ds(i * B, B), :],
                              sem.at[i]).wait()
        deg = deg + jnp.dot(ones_b, adj_v[pl.ds(i * B, B), :],
                            preferred_element_type=jnp.float32)
    dis = jax.lax.rsqrt(deg)

    adjv = adj_v[...]
    y1 = dis * xtT
    xT = jnp.maximum(
        dis * (jnp.dot(y1, adjv, preferred_element_type=jnp.float32) + 2.0 * y1)
        + jnp.transpose(b1_ref[...]), 0.0)

    xt2T = jax.lax.dot_general(w2_ref[...], xT, _TDIMS,
                               preferred_element_type=jnp.float32)
    y2 = dis * xt2T
    x2T = jnp.maximum(
        dis * (jnp.dot(y2, adjv, preferred_element_type=jnp.float32) + 2.0 * y2)
        + jnp.transpose(b2_ref[...]), 0.0)

    x_stage[...] = jnp.transpose(x2T)
    wb = pltpu.make_async_copy(x_stage, x_out_hbm, xsem)
    wb.start()

    hT = jnp.maximum(jax.lax.dot_general(fc1w_ref[...], x2T, _TDIMS,
                                         preferred_element_type=jnp.float32)
                     + jnp.transpose(fc1b_ref[...]), 0.0)
    a2T = (jax.lax.dot_general(fc2w_ref[...], hT, _TDIMS,
                               preferred_element_type=jnp.float32)
           + jnp.transpose(fc2b_ref[...]))
    a2_out_ref[...] = jnp.transpose(a2T)
    wb.wait()


def kernel(adj, node_emb, W1, b1, W2, b2, fc1_W, fc1_b, fc2_W, fc2_b):
    vmem = pl.BlockSpec(memory_space=pltpu.MemorySpace.VMEM)
    x, a2 = pl.pallas_call(
        _fused_kernel,
        in_specs=[pl.BlockSpec(memory_space=pl.ANY)] + [vmem] * 9,
        out_shape=(
            jax.ShapeDtypeStruct((N, 128), jnp.float32),
            jax.ShapeDtypeStruct((N, 1), jnp.float32),
        ),
        out_specs=(pl.BlockSpec(memory_space=pl.ANY),
                   pl.BlockSpec(memory_space=pltpu.MemorySpace.VMEM)),
        scratch_shapes=[
            pltpu.VMEM((N, N), jnp.float32),
            pltpu.VMEM((N, 128), jnp.float32),
            pltpu.SemaphoreType.DMA((NB,)),
            pltpu.SemaphoreType.DMA,
        ],
    )(adj, node_emb, W1, b1.reshape(1, 512), W2, b2.reshape(1, 128),
      fc1_W, fc1_b.reshape(1, 256), fc2_W, fc2_b.reshape(1, 1))
    return (x, a2)


# speedup vs baseline: 1.0437x; 1.0437x over previous
import jax
import jax.numpy as jnp
from jax.experimental import pallas as pl
from jax.experimental.pallas import tpu as pltpu

N = 1024
NB = 4
B = N // NB

_TDIMS = (((0,), (0,)), ((), ()))


def _fused_kernel(adj_hbm, emb_ref, w1_ref, b1_ref, w2_ref, b2_ref,
                  fc1w_ref, fc1b_ref, fc2w_ref, fc2b_ref,
                  x_out_ref, a2_out_ref, adj_v, sem):
    for i in range(NB):
        pltpu.make_async_copy(adj_hbm.at[pl.ds(i * B, B), :],
                              adj_v.at[pl.ds(i * B, B), :],
                              sem.at[i]).start()

    embT = jnp.transpose(emb_ref[...])
    xtT = jax.lax.dot_general(w1_ref[...], embT, _TDIMS,
                              preferred_element_type=jnp.float32)

    ones_b = jnp.ones((1, B), jnp.float32)
    deg = jnp.full((1, N), 2.0, jnp.float32)
    for i in range(NB):
        pltpu.make_async_copy(adj_hbm.at[pl.ds(i * B, B), :],
                              adj_v.at[pl.ds(i * B, B), :],
                              sem.at[i]).wait()
        deg = deg + jnp.dot(ones_b, adj_v[pl.ds(i * B, B), :],
                            preferred_element_type=jnp.float32)
    dis = jax.lax.rsqrt(deg)

    adjv = adj_v[...]
    y1 = dis * xtT
    xT = jnp.maximum(
        dis * (jnp.dot(y1, adjv, preferred_element_type=jnp.float32) + 2.0 * y1)
        + jnp.transpose(b1_ref[...]), 0.0)

    xt2T = jax.lax.dot_general(w2_ref[...], xT, _TDIMS,
                               preferred_element_type=jnp.float32)
    y2 = dis * xt2T
    x2T = jnp.maximum(
        dis * (jnp.dot(y2, adjv, preferred_element_type=jnp.float32) + 2.0 * y2)
        + jnp.transpose(b2_ref[...]), 0.0)
    x_out_ref[...] = jnp.transpose(x2T)

    hT = jnp.maximum(jax.lax.dot_general(fc1w_ref[...], x2T, _TDIMS,
                                         preferred_element_type=jnp.float32)
                     + jnp.transpose(fc1b_ref[...]), 0.0)
    a2T = (jax.lax.dot_general(fc2w_ref[...], hT, _TDIMS,
                               preferred_element_type=jnp.float32)
           + jnp.transpose(fc2b_ref[...]))
    a2_out_ref[...] = jnp.transpose(a2T)


def kernel(adj, node_emb, W1, b1, W2, b2, fc1_W, fc1_b, fc2_W, fc2_b):
    vmem = pl.BlockSpec(memory_space=pltpu.MemorySpace.VMEM)
    x, a2 = pl.pallas_call(
        _fused_kernel,
        in_specs=[pl.BlockSpec(memory_space=pl.ANY)] + [vmem] * 9,
        out_shape=(
            jax.ShapeDtypeStruct((N, 128), jnp.float32),
            jax.ShapeDtypeStruct((N, 1), jnp.float32),
        ),
        scratch_shapes=[
            pltpu.VMEM((N, N), jnp.float32),
            pltpu.SemaphoreType.DMA((NB,)),
        ],
    )(adj, node_emb, W1, b1.reshape(1, 512), W2, b2.reshape(1, 128),
      fc1_W, fc1_b.reshape(1, 256), fc2_W, fc2_b.reshape(1, 1))
    return (x, a2)


# speedup vs baseline: 1.1404x; 1.0926x over previous
import jax
import jax.numpy as jnp
from jax.experimental import pallas as pl
from jax.experimental.pallas import tpu as pltpu

N = 1024
NB = 4
B = N // NB

_TDIMS = (((0,), (0,)), ((), ()))


def _fused_kernel(adj_hbm, emb_hbm, w1_hbm, b1_hbm, w2_hbm, b2_hbm,
                  fc1w_hbm, fc1b_hbm, fc2w_hbm, fc2b_hbm,
                  x_out_ref, a2_out_ref,
                  adj_v, emb_v, w1_v, b1_v, w2_v, b2_v,
                  fc1w_v, fc1b_v, fc2w_v, fc2b_v, sem):
    def copy(src, dst, i):
        return pltpu.make_async_copy(src, dst, sem.at[i])

    copy(emb_hbm, emb_v, NB).start()
    copy(w1_hbm, w1_v, NB + 1).start()
    for i in range(NB):
        copy(adj_hbm.at[pl.ds(i * B, B), :], adj_v.at[pl.ds(i * B, B), :], i).start()
    cold = [(b1_hbm, b1_v), (w2_hbm, w2_v), (b2_hbm, b2_v), (fc1w_hbm, fc1w_v),
            (fc1b_hbm, fc1b_v), (fc2w_hbm, fc2w_v), (fc2b_hbm, fc2b_v)]
    for j, (s, d) in enumerate(cold):
        copy(s, d, NB + 2 + j).start()

    copy(emb_hbm, emb_v, NB).wait()
    copy(w1_hbm, w1_v, NB + 1).wait()
    embT = jnp.transpose(emb_v[...])
    xtT = jax.lax.dot_general(w1_v[...], embT, _TDIMS,
                              preferred_element_type=jnp.float32)

    ones_b = jnp.ones((1, B), jnp.float32)
    deg = jnp.full((1, N), 2.0, jnp.float32)
    for i in range(NB):
        copy(adj_hbm.at[pl.ds(i * B, B), :], adj_v.at[pl.ds(i * B, B), :], i).wait()
        deg = deg + jnp.dot(ones_b, adj_v[pl.ds(i * B, B), :],
                            preferred_element_type=jnp.float32)
    dis = jax.lax.rsqrt(deg)

    for j, (s, d) in enumerate(cold):
        copy(s, d, NB + 2 + j).wait()

    adjv = adj_v[...]
    y1 = dis * xtT
    xT = jnp.maximum(
        dis * (jnp.dot(y1, adjv, preferred_element_type=jnp.float32) + 2.0 * y1)
        + jnp.transpose(b1_v[...]), 0.0)

    xt2T = jax.lax.dot_general(w2_v[...], xT, _TDIMS,
                               preferred_element_type=jnp.float32)
    y2 = dis * xt2T
    x2T = jnp.maximum(
        dis * (jnp.dot(y2, adjv, preferred_element_type=jnp.float32) + 2.0 * y2)
        + jnp.transpose(b2_v[...]), 0.0)
    x_out_ref[...] = jnp.transpose(x2T)

    hT = jnp.maximum(jax.lax.dot_general(fc1w_v[...], x2T, _TDIMS,
                                         preferred_element_type=jnp.float32)
                     + jnp.transpose(fc1b_v[...]), 0.0)
    a2T = (jax.lax.dot_general(fc2w_v[...], hT, _TDIMS,
                               preferred_element_type=jnp.float32)
           + jnp.transpose(fc2b_v[...]))
    a2_out_ref[...] = jnp.transpose(a2T)


def kernel(adj, node_emb, W1, b1, W2, b2, fc1_W, fc1_b, fc2_W, fc2_b):
    x, a2 = pl.pallas_call(
        _fused_kernel,
        in_specs=[pl.BlockSpec(memory_space=pl.ANY)] * 10,
        out_shape=(
            jax.ShapeDtypeStruct((N, 128), jnp.float32),
            jax.ShapeDtypeStruct((N, 1), jnp.float32),
        ),
        scratch_shapes=[
            pltpu.VMEM((N, N), jnp.float32),
            pltpu.VMEM((N, 128), jnp.float32),
            pltpu.VMEM((128, 512), jnp.float32),
            pltpu.VMEM((1, 512), jnp.float32),
            pltpu.VMEM((512, 128), jnp.float32),
            pltpu.VMEM((1, 128), jnp.float32),
            pltpu.VMEM((128, 256), jnp.float32),
            pltpu.VMEM((1, 256), jnp.float32),
            pltpu.VMEM((256, 1), jnp.float32),
            pltpu.VMEM((1, 1), jnp.float32),
            pltpu.SemaphoreType.DMA((NB + 9,)),
        ],
    )(adj, node_emb, W1, b1.reshape(1, 512), W2, b2.reshape(1, 128),
      fc1_W, fc1_b.reshape(1, 256), fc2_W, fc2_b.reshape(1, 1))
    return (x, a2)
